# baseline (speedup 1.0000x reference)
import functools

import jax
import jax.numpy as jnp
from jax import lax
from jax.experimental import pallas as pl
from jax.experimental.pallas import tpu as pltpu
from jax.experimental.pallas import tpu_sc as plsc

_TOPK = 8
_EPS = 1e-10
_R, _C = 64, 8192
_GW = 128
_G = _C // _GW
_L = 16
_NC, _NS = 2, 16
_NW = _NC * _NS
_RPW = _R // _NW


def _gumbel_const(shape, dtype):
    u = jax.random.uniform(jax.random.key(1), shape, dtype=dtype)
    return -jnp.log(_EPS - jnp.log(u + _EPS))


def _dense_stage_kernel(x_ref, g_ref, z_ref, m_ref):
    z = x_ref[...] + g_ref[...]
    z_ref[...] = z
    m_ref[...] = jnp.max(z.reshape(_R, _G, _GW), axis=2)


def _iota16():
    return lax.iota(jnp.int32, _L)


def _sortd(v):
    return plsc.sort_key_val(v, _iota16(), descending=True)[0]


def _top16_desc(chunks):
    vs = [_sortd(c) for c in chunks]
    while len(vs) > 1:
        nxt = [_sortd(jnp.maximum(vs[i], lax.rev(vs[i + 1], (0,))))
               for i in range(0, len(vs) - 1, 2)]
        if len(vs) % 2:
            nxt.append(vs[-1])
        vs = nxt
    return vs[0]


def _nth_desc(t16, k):
    return jnp.min(jnp.where(_iota16() == k, t16, jnp.float32(jnp.inf)))


def _tie_cutoff(vchunks, pchunks, t, k):
    cntv = jnp.zeros((_L,), jnp.int32)
    for v in vchunks:
        cntv = cntv + jnp.where(v > t, 1, 0)
    cnt = jnp.sum(cntv)
    big = jnp.int32(2 ** 30)

    def cond(c):
        return c[0] > 0

    def body(c):
        needed, cmax = c
        best = big
        for v, p in zip(vchunks, pchunks):
            best = jnp.minimum(
                best, jnp.min(jnp.where((v == t) & (p > cmax), p, big)))
        return needed - 1, best

    _, cmax = lax.while_loop(cond, body, (k - cnt, jnp.int32(-1)))
    return cmax


_sc_mesh = plsc.VectorSubcoreMesh(
    core_axis_name="c", subcore_axis_name="s",
    num_cores=_NC, num_subcores=_NS)


_SC_SCRATCH = [
    pltpu.VMEM((_G,), jnp.float32),
    pltpu.VMEM((_L,), jnp.int32),
    pltpu.VMEM((_L, _GW), jnp.float32),
    pltpu.VMEM((_C,), jnp.float32),
    pltpu.VMEM((_L,), jnp.int32),
    pltpu.SemaphoreType.DMA,
]


def _sc_select_body(zv_hbm, m_hbm, out_hbm,
                      mrow_v, gidx_v, cand_v, rowbuf_v, selcol_v, sem):
    wid = lax.axis_index("s") * _NC + lax.axis_index("c")
    base = wid * _RPW
    iota = _iota16()
    onesf = jnp.ones((_L,), jnp.float32)
    zerosf = jnp.zeros((_L,), jnp.float32)

    def _zbody(i, carry):
        for u in range(8):
            rowbuf_v[pl.ds(i * 128 + u * 16, _L)] = zerosf
        return carry

    lax.fori_loop(0, _C // 128, _zbody, 0)

    for rr in range(_RPW):
        row = base + rr
        pltpu.sync_copy(m_hbm.at[row], mrow_v)

        mch = [mrow_v[pl.ds(_L * j, _L)] for j in range(_G // _L)]
        gch = [iota + _L * j for j in range(_G // _L)]
        tg = _nth_desc(_top16_desc(mch), _TOPK - 1)
        cmax_g = _tie_cutoff(mch, gch, tg, _TOPK)
        cbase = jnp.int32(0)
        for v, p in zip(mch, gch):
            selm = (v > tg) | ((v == tg) & (p <= cmax_g))
            seli = jnp.where(selm, 1, 0)
            ranks = cbase + plsc.cumsum(seli) - 1
            plsc.store_scatter(gidx_v, [ranks], row * _G + p, mask=selm)
            plsc.store_scatter(gidx_v, [ranks + _TOPK], row * _G + p, mask=selm)
            cbase = cbase + jnp.sum(seli)

        gidvec = jnp.clip(gidx_v[...], 0, _R * _G - 1)
        gidx_v[...] = gidvec
        pltpu.async_copy(zv_hbm.at[gidx_v], cand_v, sem).wait()

        vch, pch = [], []
        gidvec = gidvec - row * _G
        for pg in range(_TOPK):
            colbase = gidvec[pg] * _GW
            for q in range(_GW // _L):
                vch.append(cand_v[pg, pl.ds(q * _L, _L)])
                pch.append(colbase + q * _L + iota)
        t8 = _nth_desc(_top16_desc(vch), _TOPK - 1)
        cmax_c = _tie_cutoff(vch, pch, t8, _TOPK)
        cbase = jnp.int32(0)
        for v, p in zip(vch, pch):
            selc = (v > t8) | ((v == t8) & (p <= cmax_c))
            seli = jnp.where(selc, 1, 0)
            plsc.store_scatter(rowbuf_v, [p], onesf, mask=selc)
            ranks = cbase + plsc.cumsum(seli) - 1
            plsc.store_scatter(selcol_v, [ranks], p, mask=selc)
            plsc.store_scatter(selcol_v, [ranks + _TOPK], p, mask=selc)
            cbase = cbase + jnp.sum(seli)

        pltpu.sync_copy(rowbuf_v, out_hbm.at[row])
        plsc.store_scatter(rowbuf_v, [selcol_v[...]], zerosf)


def _sc_select_kernel(zv, m):
    return pl.kernel(
        _sc_select_body,
        out_type=jax.ShapeDtypeStruct((_R, _C), jnp.float32),
        mesh=_sc_mesh,
        compiler_params=pltpu.CompilerParams(needs_layout_passes=False),
        scratch_types=_SC_SCRATCH,
    )(zv, m)


def kernel(x):
    g = _gumbel_const(x.shape, x.dtype)
    z, m = pl.pallas_call(
        _dense_stage_kernel,
        out_shape=(jax.ShapeDtypeStruct((_R, _C), jnp.float32),
                   jax.ShapeDtypeStruct((_R, _G), jnp.float32)),
    )(x, g)
    return _sc_select_kernel(z.reshape(_R * _G, _GW), m)


# speedup vs baseline: 1.0097x; 1.0097x over previous
import functools

import jax
import jax.numpy as jnp
from jax import lax
from jax.experimental import pallas as pl
from jax.experimental.pallas import tpu as pltpu
from jax.experimental.pallas import tpu_sc as plsc

_TOPK = 8
_EPS = 1e-10
_R, _C = 64, 8192
_GW = 128
_G = _C // _GW
_L = 16
_NC, _NS = 2, 16
_NW = _NC * _NS
_RPW = _R // _NW


def _gumbel_const(shape, dtype):
    u = jax.random.uniform(jax.random.key(1), shape, dtype=dtype)
    return -jnp.log(_EPS - jnp.log(u + _EPS))


def _dense_stage_kernel(x_ref, g_ref, z_ref, m_ref):
    z = x_ref[...] + g_ref[...]
    z4 = z.reshape(8, 8, _G, _GW).transpose(0, 2, 1, 3).reshape(_R * _G, _GW)
    z_ref[...] = z4
    m_ref[...] = jnp.max(z4, axis=1)


def _iota16():
    return lax.iota(jnp.int32, _L)


def _sortd(v):
    return plsc.sort_key_val(v, _iota16(), descending=True)[0]


def _top16_desc(chunks):
    vs = [_sortd(c) for c in chunks]
    while len(vs) > 1:
        nxt = [_sortd(jnp.maximum(vs[i], lax.rev(vs[i + 1], (0,))))
               for i in range(0, len(vs) - 1, 2)]
        if len(vs) % 2:
            nxt.append(vs[-1])
        vs = nxt
    return vs[0]


def _nth_desc(t16, k):
    return jnp.min(jnp.where(_iota16() == k, t16, jnp.float32(jnp.inf)))


def _tie_cutoff(vchunks, pchunks, t, k):
    cntv = jnp.zeros((_L,), jnp.int32)
    for v in vchunks:
        cntv = cntv + jnp.where(v > t, 1, 0)
    cnt = jnp.sum(cntv)
    big = jnp.int32(2 ** 30)

    def cond(c):
        return c[0] > 0

    def body(c):
        needed, cmax = c
        best = big
        for v, p in zip(vchunks, pchunks):
            best = jnp.minimum(
                best, jnp.min(jnp.where((v == t) & (p > cmax), p, big)))
        return needed - 1, best

    _, cmax = lax.while_loop(cond, body, (k - cnt, jnp.int32(-1)))
    return cmax


_sc_mesh = plsc.VectorSubcoreMesh(
    core_axis_name="c", subcore_axis_name="s",
    num_cores=_NC, num_subcores=_NS)


_SC_SCRATCH = [
    pltpu.VMEM((_G,), jnp.float32),
    pltpu.VMEM((_L,), jnp.int32),
    pltpu.VMEM((_L, _GW), jnp.float32),
    pltpu.VMEM((_C,), jnp.float32),
    pltpu.VMEM((_L,), jnp.int32),
    pltpu.SemaphoreType.DMA,
]


def _sc_select_body(zv_hbm, m_hbm, out_hbm,
                      mrow_v, gidx_v, cand_v, rowbuf_v, selcol_v, sem):
    wid = lax.axis_index("s") * _NC + lax.axis_index("c")
    base = wid * _RPW
    iota = _iota16()
    onesf = jnp.ones((_L,), jnp.float32)
    zerosf = jnp.zeros((_L,), jnp.float32)

    def _zbody(i, carry):
        for u in range(8):
            rowbuf_v[pl.ds(i * 128 + u * 16, _L)] = zerosf
        return carry

    lax.fori_loop(0, _C // 128, _zbody, 0)

    for rr in range(_RPW):
        row = base + rr
        pltpu.sync_copy(m_hbm.at[row], mrow_v)

        mch = [mrow_v[pl.ds(_L * j, _L)] for j in range(_G // _L)]
        gch = [iota + _L * j for j in range(_G // _L)]
        tg = _nth_desc(_top16_desc(mch), _TOPK - 1)
        cmax_g = _tie_cutoff(mch, gch, tg, _TOPK)
        kbase = (row // 8) * (_G * 8) + row % 8
        cbase = jnp.int32(0)
        for v, p in zip(mch, gch):
            selm = (v > tg) | ((v == tg) & (p <= cmax_g))
            seli = jnp.where(selm, 1, 0)
            ranks = cbase + plsc.cumsum(seli) - 1
            plsc.store_scatter(gidx_v, [ranks], kbase + p * 8, mask=selm)
            plsc.store_scatter(gidx_v, [ranks + _TOPK], kbase + p * 8, mask=selm)
            cbase = cbase + jnp.sum(seli)

        gidvec = jnp.clip(gidx_v[...], 0, _R * _G - 1)
        gidx_v[...] = gidvec
        pltpu.async_copy(zv_hbm.at[gidx_v], cand_v, sem).wait()

        vch, pch = [], []
        gidvec = (gidvec >> 3) & (_G - 1)
        for pg in range(_TOPK):
            colbase = gidvec[pg] * _GW
            for q in range(_GW // _L):
                vch.append(cand_v[pg, pl.ds(q * _L, _L)])
                pch.append(colbase + q * _L + iota)
        t8 = _nth_desc(_top16_desc(vch), _TOPK - 1)
        cmax_c = _tie_cutoff(vch, pch, t8, _TOPK)
        cbase = jnp.int32(0)
        for v, p in zip(vch, pch):
            selc = (v > t8) | ((v == t8) & (p <= cmax_c))
            seli = jnp.where(selc, 1, 0)
            plsc.store_scatter(rowbuf_v, [p], onesf, mask=selc)
            ranks = cbase + plsc.cumsum(seli) - 1
            plsc.store_scatter(selcol_v, [ranks], p, mask=selc)
            plsc.store_scatter(selcol_v, [ranks + _TOPK], p, mask=selc)
            cbase = cbase + jnp.sum(seli)

        pltpu.sync_copy(rowbuf_v, out_hbm.at[row])
        plsc.store_scatter(rowbuf_v, [selcol_v[...]], zerosf)


def _sc_select_kernel(zv, m):
    return pl.kernel(
        _sc_select_body,
        out_type=jax.ShapeDtypeStruct((_R, _C), jnp.float32),
        mesh=_sc_mesh,
        compiler_params=pltpu.CompilerParams(needs_layout_passes=False),
        scratch_types=_SC_SCRATCH,
    )(zv, m)


def kernel(x):
    g = _gumbel_const(x.shape, x.dtype)
    z4, m4 = pl.pallas_call(
        _dense_stage_kernel,
        out_shape=(jax.ShapeDtypeStruct((_R * _G, _GW), jnp.float32),
                   jax.ShapeDtypeStruct((_R * _G,), jnp.float32)),
    )(x, g)
    m = m4.reshape(8, _G, 8).transpose(0, 2, 1).reshape(_R, _G)
    return _sc_select_kernel(z4, m)
